# baseline (speedup 1.0000x reference)
import jax
import jax.numpy as jnp
from jax import lax
from jax.experimental import pallas as pl
from jax.experimental.pallas import tpu as pltpu
from jax.experimental.pallas import tpu_sc as plsc

NC = 2
NS = 16
L = 16
NW = NC * NS
B = 16384
D = 32
PK = 128 // D
BPW = B // NW
CHUNK = 128
NCH = BPW // CHUNK
GPC = CHUNK // L
NBUF = 2


def _sc_kernel(users_hbm, pos_hbm, neg_hbm, ue_hbm, ie_hbm, ub_hbm, ib_hbm,
               outp_hbm, outn_hbm,
               uidx, pidx, nidx, uq, pq, nq, ue_rows, pe_rows, ne_rows,
               ub_v, pb_v, nb_v, outp_v, outn_v, sem, bsem):
    wid = lax.axis_index("core") * NS + lax.axis_index("subcore")

    pltpu.sync_copy(users_hbm.at[wid], uidx)
    pltpu.sync_copy(pos_hbm.at[wid], pidx)
    pltpu.sync_copy(neg_hbm.at[wid], nidx)

    for j in range(NCH):
        for t in range(GPC):
            sl = pl.ds(t * L, L)
            uq[j, sl] = uidx[j, sl] >> 2
            pq[j, sl] = pidx[j, sl] >> 2
            nq[j, sl] = nidx[j, sl] >> 2

    bcopies = []
    for j in range(NCH):
        sl = pl.ds(j * CHUNK, CHUNK)
        bcopies.append(pltpu.async_copy(ub_hbm.at[uidx.at[j]], ub_v.at[sl], bsem))
        bcopies.append(pltpu.async_copy(ib_hbm.at[pidx.at[j]], pb_v.at[sl], bsem))
        bcopies.append(pltpu.async_copy(ib_hbm.at[nidx.at[j]], nb_v.at[sl], bsem))

    def fire(j):
        b = j % NBUF
        sl = pl.ds(b * CHUNK, CHUNK)
        return [pltpu.async_copy(ue_hbm.at[uq.at[j]], ue_rows.at[sl], sem),
                pltpu.async_copy(ie_hbm.at[pq.at[j]], pe_rows.at[sl], sem),
                pltpu.async_copy(ie_hbm.at[nq.at[j]], ne_rows.at[sl], sem)]

    copies = []
    for j in range(NBUF):
        copies.append(fire(j))

    for c in bcopies:
        c.wait()

    lanes = lax.iota(jnp.int32, L)

    for j in range(NCH):
        for c in copies[j]:
            c.wait()
        b = j % NBUF

        @pl.loop(0, GPC)
        def _group(g, j=j, b=b):
            pos = b * CHUNK + g * L + lanes
            csl = pl.ds(g * L, L)
            ush = (uidx[j, csl] & (PK - 1)) << 5
            psh = (pidx[j, csl] & (PK - 1)) << 5
            nsh = (nidx[j, csl] & (PK - 1)) << 5
            accp = jnp.zeros((L,), jnp.float32)
            accn = jnp.zeros((L,), jnp.float32)
            for dd in range(D):
                rot = (lanes + dd) & (D - 1)
                u = plsc.load_gather(ue_rows, [pos, ush + rot])
                p = plsc.load_gather(pe_rows, [pos, psh + rot])
                n = plsc.load_gather(ne_rows, [pos, nsh + rot])
                accp = accp + u * p
                accn = accn + u * n
            osl = pl.ds(j * CHUNK + g * L, L)
            ub = ub_v[osl]
            outp_v[osl] = accp + ub + pb_v[osl]
            outn_v[osl] = accn + ub + nb_v[osl]

        if j + NBUF < NCH:
            copies.append(fire(j + NBUF))

    pltpu.sync_copy(outp_v, outp_hbm.at[pl.ds(wid * BPW, BPW)])
    pltpu.sync_copy(outn_v, outn_hbm.at[pl.ds(wid * BPW, BPW)])


def kernel(users, pos_items, neg_items, user_embeddings, item_embeddings,
           user_biases, item_biases):
    users3 = users.astype(jnp.int32).reshape(NW, NCH, CHUNK)
    pos3 = pos_items.astype(jnp.int32).reshape(NW, NCH, CHUNK)
    neg3 = neg_items.astype(jnp.int32).reshape(NW, NCH, CHUNK)
    nrow = user_embeddings.shape[0]
    ue4 = user_embeddings.reshape(nrow // PK, PK * D)
    ie4 = item_embeddings.reshape(nrow // PK, PK * D)
    ub1 = user_biases.reshape(-1)
    ib1 = item_biases.reshape(-1)

    mesh = plsc.VectorSubcoreMesh(core_axis_name="core",
                                  subcore_axis_name="subcore",
                                  num_cores=NC, num_subcores=NS)
    f = pl.kernel(
        _sc_kernel,
        compiler_params=pltpu.CompilerParams(needs_layout_passes=False,
                                             use_tc_tiling_on_sc=False),
        out_type=(jax.ShapeDtypeStruct((B,), jnp.float32),
                  jax.ShapeDtypeStruct((B,), jnp.float32)),
        mesh=mesh,
        scratch_types=[
            pltpu.VMEM((NCH, CHUNK), jnp.int32),
            pltpu.VMEM((NCH, CHUNK), jnp.int32),
            pltpu.VMEM((NCH, CHUNK), jnp.int32),
            pltpu.VMEM((NCH, CHUNK), jnp.int32),
            pltpu.VMEM((NCH, CHUNK), jnp.int32),
            pltpu.VMEM((NCH, CHUNK), jnp.int32),
            pltpu.VMEM((NBUF * CHUNK, PK * D), jnp.float32),
            pltpu.VMEM((NBUF * CHUNK, PK * D), jnp.float32),
            pltpu.VMEM((NBUF * CHUNK, PK * D), jnp.float32),
            pltpu.VMEM((BPW,), jnp.float32),
            pltpu.VMEM((BPW,), jnp.float32),
            pltpu.VMEM((BPW,), jnp.float32),
            pltpu.VMEM((BPW,), jnp.float32),
            pltpu.VMEM((BPW,), jnp.float32),
            pltpu.SemaphoreType.DMA,
            pltpu.SemaphoreType.DMA,
        ],
    )
    pos_preds, neg_preds = f(users3, pos3, neg3, ue4, ie4, ub1, ib1)
    return pos_preds, neg_preds
